# Initial kernel scaffold; baseline (speedup 1.0000x reference)
import jax
import jax.numpy as jnp
from jax.experimental import pallas as pl


def kernel(x, gate_w, gate_b, fc1_w, fc1_b, fc2_w, fc2_b):
    raise NotImplementedError("write your pallas kernel here")


# speedup vs baseline: 3.5117x; 3.5117x over previous
import functools

import jax
import jax.numpy as jnp
from jax import lax
from jax.experimental import pallas as pl
from jax.experimental.pallas import tpu as pltpu
from jax.experimental.pallas import tpu_sc as plsc

B, S, H = 4, 8192, 1024
FFN = 512
E = 16
N = B * S
BT = 256
GBLK = (2 * N) // BT + E
P = GBLK * BT

NW = 32
TW = N // NW
CT = 32
NCH = TW // CT


def _router_body(x_ref, gw_ref, gb_ref, s0_ref, s1_ref, w0_ref, w1_ref):
    x = x_ref[...]
    logits = lax.dot_general(x, gw_ref[...], (((1,), (1,)), ((), ())),
                             preferred_element_type=jnp.float32)
    logits = logits + gb_ref[...]
    lane = lax.broadcasted_iota(jnp.int32, logits.shape, 1)
    m1 = jnp.max(logits, axis=1, keepdims=True)
    a1 = jnp.min(jnp.where(logits == m1, lane, E), axis=1, keepdims=True)
    masked = jnp.where(lane == a1, -1e30, logits)
    m2 = jnp.max(masked, axis=1, keepdims=True)
    a2 = jnp.min(jnp.where(masked == m2, lane, E), axis=1, keepdims=True)
    w1 = 1.0 / (1.0 + jnp.exp(m2 - m1))
    s0_ref[...] = a1
    s1_ref[...] = a2
    w0_ref[...] = jnp.broadcast_to(w1, (w1.shape[0], 16))
    w1_ref[...] = jnp.broadcast_to(1.0 - w1, (w1.shape[0], 16))


def _router(x_flat, gate_w, gate_b):
    tb = 1024
    out = jax.ShapeDtypeStruct((N, 1), jnp.int32)
    outf = jax.ShapeDtypeStruct((N, 16), jnp.float32)
    return pl.pallas_call(
        _router_body,
        grid=(N // tb,),
        in_specs=[
            pl.BlockSpec((tb, H), lambda g: (g, 0)),
            pl.BlockSpec((E, H), lambda g: (0, 0)),
            pl.BlockSpec((1, E), lambda g: (0, 0)),
        ],
        out_specs=[pl.BlockSpec((tb, 1), lambda g: (g, 0))] * 2
        + [pl.BlockSpec((tb, 16), lambda g: (g, 0))] * 2,
        out_shape=[out, out, outf, outf],
    )(x_flat, gate_w, gate_b.reshape(1, E))


def _sortmeta_body(a_ref, dest_ref, be_ref):
    a = a_ref[...]
    r1 = lax.broadcasted_iota(jnp.int32, (128, 128), 0)
    c1 = lax.broadcasted_iota(jnp.int32, (128, 128), 1)
    u128 = (r1 < c1).astype(jnp.float32)
    r5 = lax.broadcasted_iota(jnp.int32, (512, 512), 0)
    c5 = lax.broadcasted_iota(jnp.int32, (512, 512), 1)
    s512 = (r5 > c5).astype(jnp.float32)
    gstart = (lax.broadcasted_iota(jnp.int32, (1, 512), 1)
              .astype(jnp.float32) * BT)

    dest = jnp.zeros((512, 128), jnp.float32)
    nactive = jnp.zeros((1, 512), jnp.float32)
    off = jnp.float32(0.0)
    for e in range(E):
        oh = (a == e).astype(jnp.float32)
        rowpre = lax.dot_general(oh, u128, (((1,), (0,)), ((), ())),
                                 preferred_element_type=jnp.float32,
                                 precision=lax.Precision.HIGHEST)
        rt = jnp.sum(oh, axis=1, keepdims=True)
        rtb = jnp.broadcast_to(rt, (512, 128))
        blkpre = lax.dot_general(s512, rtb, (((1,), (0,)), ((), ())),
                                 preferred_element_type=jnp.float32,
                                 precision=lax.Precision.HIGHEST)
        rank = rowpre + blkpre
        cnt = jnp.sum(rt)
        dest = jnp.where(oh > 0, off + rank, dest)
        nactive = nactive + (gstart >= off).astype(jnp.float32)
        off = off + jnp.ceil(cnt / BT) * BT
    dest_ref[...] = dest.astype(jnp.int32)
    be = jnp.clip(nactive - 1.0, 0.0, float(E - 1))
    be_ref[...] = be.astype(jnp.int32)


def _sortmeta(sel01):
    return pl.pallas_call(
        _sortmeta_body,
        out_shape=[jax.ShapeDtypeStruct((512, 128), jnp.int32),
                   jax.ShapeDtypeStruct((1, 512), jnp.int32)],
    )(sel01)


def _dispatch_body(x_hbm, d0_hbm, d1_hbm, xs_hbm,
                   d0_v, d1_v, rows_v, sem0, sem1):
    w = lax.axis_index("s") * 2 + lax.axis_index("c")
    pltpu.sync_copy(d0_hbm.at[w], d0_v)
    pltpu.sync_copy(d1_hbm.at[w], d1_v)

    @pl.loop(0, NCH)
    def _chunk(c):
        base = w * TW + c * CT
        pltpu.sync_copy(x_hbm.at[pl.ds(base, CT)], rows_v)
        cp0 = pltpu.make_async_copy(rows_v, xs_hbm.at[d0_v.at[c]], sem0)
        cp1 = pltpu.make_async_copy(rows_v, xs_hbm.at[d1_v.at[c]], sem1)
        cp0.start()
        cp1.start()
        cp0.wait()
        cp1.wait()


def _dispatch(x_flat, d0, d1):
    mesh = plsc.VectorSubcoreMesh(core_axis_name="c", subcore_axis_name="s")
    kern = pl.kernel(
        _dispatch_body,
        out_type=jax.ShapeDtypeStruct((P, H), jnp.float32),
        mesh=mesh,
        scratch_types=[
            pltpu.VMEM((NCH, CT), jnp.int32),
            pltpu.VMEM((NCH, CT), jnp.int32),
            pltpu.VMEM((CT, H), jnp.float32),
            pltpu.SemaphoreType.DMA,
            pltpu.SemaphoreType.DMA,
        ],
    )
    return kern(x_flat, d0.reshape(NW, NCH, CT), d1.reshape(NW, NCH, CT))


_A1, _A2, _A3, _A4, _A5 = (0.254829592, -0.284496736, 1.421413741,
                           -1.453152027, 1.061405429)


def _gelu_exact(x):
    z = x * 0.7071067811865476
    az = jnp.abs(z)
    t = 1.0 / (1.0 + 0.3275911 * az)
    poly = t * (_A1 + t * (_A2 + t * (_A3 + t * (_A4 + t * _A5))))
    erf_az = 1.0 - poly * jnp.exp(-az * az)
    erf_z = jnp.where(z < 0, -erf_az, erf_az)
    return 0.5 * x * (1.0 + erf_z)


def _ffn_body(be_ref, xs_ref, f1_ref, b1_ref, f2_ref, b2_ref, ys_ref):
    xb = xs_ref[...]
    h = lax.dot_general(xb, f1_ref[0], (((1,), (1,)), ((), ())),
                        preferred_element_type=jnp.float32)
    h = _gelu_exact(h + b1_ref[0])
    y = lax.dot_general(h, f2_ref[0], (((1,), (1,)), ((), ())),
                        preferred_element_type=jnp.float32)
    ys_ref[...] = y + b2_ref[0]


def _ffn(xs, block_expert, fc1_w, fc1_b, fc2_w, fc2_b):
    grid_spec = pltpu.PrefetchScalarGridSpec(
        num_scalar_prefetch=1,
        grid=(GBLK,),
        in_specs=[
            pl.BlockSpec((BT, H), lambda g, be: (g, 0)),
            pl.BlockSpec((1, FFN, H), lambda g, be: (be[g], 0, 0)),
            pl.BlockSpec((1, 1, FFN), lambda g, be: (be[g], 0, 0)),
            pl.BlockSpec((1, H, FFN), lambda g, be: (be[g], 0, 0)),
            pl.BlockSpec((1, 1, H), lambda g, be: (be[g], 0, 0)),
        ],
        out_specs=pl.BlockSpec((BT, H), lambda g, be: (g, 0)),
    )
    return pl.pallas_call(
        _ffn_body,
        grid_spec=grid_spec,
        out_shape=jax.ShapeDtypeStruct((P, H), jnp.float32),
    )(block_expert, xs, fc1_w, fc1_b.reshape(E, 1, FFN),
      fc2_w, fc2_b.reshape(E, 1, H))


def _combine_body(ys_hbm, d0_hbm, d1_hbm, w0_hbm, w1_hbm, out_hbm,
                  d0_v, d1_v, w0_v, w1_v, r0_v, r1_v, o_v, sem0, sem1):
    w = lax.axis_index("s") * 2 + lax.axis_index("c")
    pltpu.sync_copy(d0_hbm.at[w], d0_v)
    pltpu.sync_copy(d1_hbm.at[w], d1_v)

    @pl.loop(0, NCH)
    def _chunk(c):
        cp0 = pltpu.make_async_copy(ys_hbm.at[d0_v.at[c]], r0_v, sem0)
        cp1 = pltpu.make_async_copy(ys_hbm.at[d1_v.at[c]], r1_v, sem1)
        cp0.start()
        cp1.start()
        pltpu.sync_copy(w0_hbm.at[w, c], w0_v)
        pltpu.sync_copy(w1_hbm.at[w, c], w1_v)
        cp0.wait()
        cp1.wait()

        @pl.loop(0, CT)
        def _tok(t):
            wv0 = w0_v[t]
            wv1 = w1_v[t]

            @pl.loop(0, H // 16)
            def _vec(j):
                sl = pl.ds(j * 16, 16)
                o_v[t, sl] = wv0 * r0_v[t, sl] + wv1 * r1_v[t, sl]

        pltpu.sync_copy(o_v, out_hbm.at[pl.ds(w * TW + c * CT, CT)])


def _combine(ys, d0, d1, w0, w1):
    mesh = plsc.VectorSubcoreMesh(core_axis_name="c", subcore_axis_name="s")
    kern = pl.kernel(
        _combine_body,
        out_type=jax.ShapeDtypeStruct((N, H), jnp.float32),
        mesh=mesh,
        scratch_types=[
            pltpu.VMEM((NCH, CT), jnp.int32),
            pltpu.VMEM((NCH, CT), jnp.int32),
            pltpu.VMEM((CT, 16), jnp.float32),
            pltpu.VMEM((CT, 16), jnp.float32),
            pltpu.VMEM((CT, H), jnp.float32),
            pltpu.VMEM((CT, H), jnp.float32),
            pltpu.VMEM((CT, H), jnp.float32),
            pltpu.SemaphoreType.DMA,
            pltpu.SemaphoreType.DMA,
        ],
    )
    return kern(ys,
                d0.reshape(NW, NCH, CT), d1.reshape(NW, NCH, CT),
                w0.reshape(NW, NCH, CT, 16), w1.reshape(NW, NCH, CT, 16))


@jax.jit
def kernel(x, gate_w, gate_b, fc1_w, fc1_b, fc2_w, fc2_b):
    x_flat = x.reshape(N, H)
    s0, s1, w0, w1 = _router(x_flat, gate_w, gate_b)
    sel01 = jnp.concatenate(
        [s0.reshape(256, 128), s1.reshape(256, 128)], axis=0)
    dest, be = _sortmeta(sel01)
    d0 = dest[:256].reshape(N)
    d1 = dest[256:].reshape(N)
    block_expert = be.reshape(512)[:GBLK]
    xs = _dispatch(x_flat, d0, d1)
    ys = _ffn(xs, block_expert, fc1_w, fc1_b, fc2_w, fc2_b)
    out = _combine(ys, d0, d1, w0, w1)
    return out.reshape(B, S, H)


# speedup vs baseline: 3.8522x; 1.0969x over previous
import jax
import jax.numpy as jnp
from jax import lax
from jax.experimental import pallas as pl
from jax.experimental.pallas import tpu as pltpu
from jax.experimental.pallas import tpu_sc as plsc

B, S, H = 4, 8192, 1024
FFN = 512
E = 16
N = B * S
BT = 256
GBLK = (2 * N) // BT + E
P = GBLK * BT

NW = 32
TW = N // NW
CTD = 32
NCHD = TW // CTD
CTC = 16
NCHC = TW // CTC


def _router_body(x_ref, gw_ref, gb_ref, s0_ref, s1_ref, w0_ref, w1_ref):
    x = x_ref[...]
    logits = lax.dot_general(x, gw_ref[...], (((1,), (1,)), ((), ())),
                             preferred_element_type=jnp.float32)
    logits = logits + gb_ref[...]
    lane = lax.broadcasted_iota(jnp.int32, logits.shape, 1)
    m1 = jnp.max(logits, axis=1, keepdims=True)
    a1 = jnp.min(jnp.where(logits == m1, lane, E), axis=1, keepdims=True)
    masked = jnp.where(lane == a1, -1e30, logits)
    m2 = jnp.max(masked, axis=1, keepdims=True)
    a2 = jnp.min(jnp.where(masked == m2, lane, E), axis=1, keepdims=True)
    w1 = 1.0 / (1.0 + jnp.exp(m2 - m1))
    s0_ref[...] = a1
    s1_ref[...] = a2
    w0_ref[...] = jnp.broadcast_to(w1, (w1.shape[0], 16))
    w1_ref[...] = jnp.broadcast_to(1.0 - w1, (w1.shape[0], 16))


def _router(x_flat, gate_w, gate_b):
    tb = 1024
    out = jax.ShapeDtypeStruct((N, 1), jnp.int32)
    outf = jax.ShapeDtypeStruct((N, 16), jnp.float32)
    return pl.pallas_call(
        _router_body,
        grid=(N // tb,),
        in_specs=[
            pl.BlockSpec((tb, H), lambda g: (g, 0)),
            pl.BlockSpec((E, H), lambda g: (0, 0)),
            pl.BlockSpec((1, E), lambda g: (0, 0)),
        ],
        out_specs=[pl.BlockSpec((tb, 1), lambda g: (g, 0))] * 2
        + [pl.BlockSpec((tb, 16), lambda g: (g, 0))] * 2,
        out_shape=[out, out, outf, outf],
    )(x_flat, gate_w, gate_b.reshape(1, E))


def _sortmeta_body(a_ref, dest_ref, be_ref):
    a = a_ref[...]
    r1 = lax.broadcasted_iota(jnp.int32, (128, 128), 0)
    c1 = lax.broadcasted_iota(jnp.int32, (128, 128), 1)
    u128 = (r1 < c1).astype(jnp.float32)
    r5 = lax.broadcasted_iota(jnp.int32, (512, 512), 0)
    c5 = lax.broadcasted_iota(jnp.int32, (512, 512), 1)
    s512 = (r5 > c5).astype(jnp.float32)
    gstart = (lax.broadcasted_iota(jnp.int32, (1, 512), 1)
              .astype(jnp.float32) * BT)

    dest = jnp.zeros((512, 128), jnp.float32)
    nactive = jnp.zeros((1, 512), jnp.float32)
    off = jnp.float32(0.0)
    for e in range(E):
        oh = (a == e).astype(jnp.float32)
        rowpre = lax.dot_general(oh, u128, (((1,), (0,)), ((), ())),
                                 preferred_element_type=jnp.float32,
                                 precision=lax.Precision.HIGHEST)
        rt = jnp.sum(oh, axis=1, keepdims=True)
        rtb = jnp.broadcast_to(rt, (512, 128))
        blkpre = lax.dot_general(s512, rtb, (((1,), (0,)), ((), ())),
                                 preferred_element_type=jnp.float32,
                                 precision=lax.Precision.HIGHEST)
        rank = rowpre + blkpre
        cnt = jnp.sum(rt)
        dest = jnp.where(oh > 0, off + rank, dest)
        nactive = nactive + (gstart >= off).astype(jnp.float32)
        off = off + jnp.ceil(cnt / BT) * BT
    dest_ref[...] = dest.astype(jnp.int32)
    be = jnp.clip(nactive - 1.0, 0.0, float(E - 1))
    be_ref[...] = be.astype(jnp.int32)


def _sortmeta(sel01):
    return pl.pallas_call(
        _sortmeta_body,
        out_shape=[jax.ShapeDtypeStruct((512, 128), jnp.int32),
                   jax.ShapeDtypeStruct((1, 512), jnp.int32)],
    )(sel01)


def _dispatch_body(x_hbm, d0_hbm, d1_hbm, xs_hbm,
                   d0_v, d1_v, buf_a, buf_b, sa0, sa1, sb0, sb1):
    w = lax.axis_index("s") * 2 + lax.axis_index("c")
    pltpu.sync_copy(d0_hbm.at[w], d0_v)
    pltpu.sync_copy(d1_hbm.at[w], d1_v)

    @pl.loop(0, NCHD // 2)
    def _pair(p):
        ca = 2 * p
        cb = 2 * p + 1
        pltpu.sync_copy(x_hbm.at[pl.ds(w * TW + ca * CTD, CTD)], buf_a)
        cpa0 = pltpu.make_async_copy(buf_a, xs_hbm.at[d0_v.at[ca]], sa0)
        cpa1 = pltpu.make_async_copy(buf_a, xs_hbm.at[d1_v.at[ca]], sa1)
        cpa0.start()
        cpa1.start()

        @pl.when(p > 0)
        def _():
            pltpu.make_async_copy(buf_b, xs_hbm.at[d0_v.at[cb]], sb0).wait()
            pltpu.make_async_copy(buf_b, xs_hbm.at[d1_v.at[cb]], sb1).wait()

        pltpu.sync_copy(x_hbm.at[pl.ds(w * TW + cb * CTD, CTD)], buf_b)
        cpa0.wait()
        cpa1.wait()
        pltpu.make_async_copy(buf_b, xs_hbm.at[d0_v.at[cb]], sb0).start()
        pltpu.make_async_copy(buf_b, xs_hbm.at[d1_v.at[cb]], sb1).start()

    last = NCHD - 1
    pltpu.make_async_copy(buf_b, xs_hbm.at[d0_v.at[last]], sb0).wait()
    pltpu.make_async_copy(buf_b, xs_hbm.at[d1_v.at[last]], sb1).wait()


def _dispatch(x_flat, d0, d1):
    mesh = plsc.VectorSubcoreMesh(core_axis_name="c", subcore_axis_name="s")
    kern = pl.kernel(
        _dispatch_body,
        out_type=jax.ShapeDtypeStruct((P, H), jnp.float32),
        mesh=mesh,
        scratch_types=[
            pltpu.VMEM((NCHD, CTD), jnp.int32),
            pltpu.VMEM((NCHD, CTD), jnp.int32),
            pltpu.VMEM((CTD, H), jnp.float32),
            pltpu.VMEM((CTD, H), jnp.float32),
            pltpu.SemaphoreType.DMA,
            pltpu.SemaphoreType.DMA,
            pltpu.SemaphoreType.DMA,
            pltpu.SemaphoreType.DMA,
        ],
    )
    return kern(x_flat, d0.reshape(NW, NCHD, CTD), d1.reshape(NW, NCHD, CTD))


_A1, _A2, _A3, _A4, _A5 = (0.254829592, -0.284496736, 1.421413741,
                           -1.453152027, 1.061405429)


def _gelu_exact(x):
    z = x * 0.7071067811865476
    az = jnp.abs(z)
    t = 1.0 / (1.0 + 0.3275911 * az)
    poly = t * (_A1 + t * (_A2 + t * (_A3 + t * (_A4 + t * _A5))))
    erf_az = 1.0 - poly * jnp.exp(-az * az)
    erf_z = jnp.where(z < 0, -erf_az, erf_az)
    return 0.5 * x * (1.0 + erf_z)


def _ffn_body(be_ref, xs_ref, f1_ref, b1_ref, f2_ref, b2_ref, ys_ref):
    xb = xs_ref[...].astype(jnp.bfloat16)
    h = lax.dot_general(xb, f1_ref[0], (((1,), (1,)), ((), ())),
                        preferred_element_type=jnp.float32)
    h = _gelu_exact(h + b1_ref[0])
    y = lax.dot_general(h.astype(jnp.bfloat16), f2_ref[0],
                        (((1,), (1,)), ((), ())),
                        preferred_element_type=jnp.float32)
    ys_ref[...] = y + b2_ref[0]


def _ffn(xs, block_expert, fc1_w, fc1_b, fc2_w, fc2_b):
    grid_spec = pltpu.PrefetchScalarGridSpec(
        num_scalar_prefetch=1,
        grid=(GBLK,),
        in_specs=[
            pl.BlockSpec((BT, H), lambda g, be: (g, 0)),
            pl.BlockSpec((1, FFN, H), lambda g, be: (be[g], 0, 0)),
            pl.BlockSpec((1, 1, FFN), lambda g, be: (be[g], 0, 0)),
            pl.BlockSpec((1, H, FFN), lambda g, be: (be[g], 0, 0)),
            pl.BlockSpec((1, 1, H), lambda g, be: (be[g], 0, 0)),
        ],
        out_specs=pl.BlockSpec((BT, H), lambda g, be: (g, 0)),
    )
    return pl.pallas_call(
        _ffn_body,
        grid_spec=grid_spec,
        out_shape=jax.ShapeDtypeStruct((P, H), jnp.float32),
    )(block_expert, xs, fc1_w.astype(jnp.bfloat16),
      fc1_b.reshape(E, 1, FFN), fc2_w.astype(jnp.bfloat16),
      fc2_b.reshape(E, 1, H))


def _combine_body(ys_hbm, d0_hbm, d1_hbm, w0_hbm, w1_hbm, out_hbm,
                  d0_v, d1_v, w0_v, w1_v,
                  ga0, ga1, gb0, gb1, o_v, sa0, sa1, sb0, sb1):
    w = lax.axis_index("s") * 2 + lax.axis_index("c")
    pltpu.sync_copy(d0_hbm.at[w], d0_v)
    pltpu.sync_copy(d1_hbm.at[w], d1_v)

    def _compute(c, g0, g1):
        @pl.loop(0, CTC)
        def _tok(t):
            wv0 = w0_v[t]
            wv1 = w1_v[t]

            @pl.loop(0, H // 16, unroll=4)
            def _vec(j):
                sl = pl.ds(j * 16, 16)
                o_v[t, sl] = wv0 * g0[t, sl] + wv1 * g1[t, sl]

        pltpu.sync_copy(o_v, out_hbm.at[pl.ds(w * TW + c * CTC, CTC)])

    pltpu.make_async_copy(ys_hbm.at[d0_v.at[0]], ga0, sa0).start()
    pltpu.make_async_copy(ys_hbm.at[d1_v.at[0]], ga1, sa1).start()

    @pl.loop(0, NCHC // 2)
    def _pair(p):
        ca = 2 * p
        cb = 2 * p + 1
        pltpu.make_async_copy(ys_hbm.at[d0_v.at[ca]], ga0, sa0).wait()
        pltpu.make_async_copy(ys_hbm.at[d1_v.at[ca]], ga1, sa1).wait()
        pltpu.make_async_copy(ys_hbm.at[d0_v.at[cb]], gb0, sb0).start()
        pltpu.make_async_copy(ys_hbm.at[d1_v.at[cb]], gb1, sb1).start()
        pltpu.sync_copy(w0_hbm.at[w, ca], w0_v)
        pltpu.sync_copy(w1_hbm.at[w, ca], w1_v)
        _compute(ca, ga0, ga1)
        pltpu.make_async_copy(ys_hbm.at[d0_v.at[cb]], gb0, sb0).wait()
        pltpu.make_async_copy(ys_hbm.at[d1_v.at[cb]], gb1, sb1).wait()

        @pl.when(p < NCHC // 2 - 1)
        def _():
            pltpu.make_async_copy(ys_hbm.at[d0_v.at[ca + 2]], ga0, sa0).start()
            pltpu.make_async_copy(ys_hbm.at[d1_v.at[ca + 2]], ga1, sa1).start()

        pltpu.sync_copy(w0_hbm.at[w, cb], w0_v)
        pltpu.sync_copy(w1_hbm.at[w, cb], w1_v)
        _compute(cb, gb0, gb1)


def _combine(ys, d0, d1, w0, w1):
    mesh = plsc.VectorSubcoreMesh(core_axis_name="c", subcore_axis_name="s")
    kern = pl.kernel(
        _combine_body,
        out_type=jax.ShapeDtypeStruct((N, H), jnp.float32),
        mesh=mesh,
        scratch_types=[
            pltpu.VMEM((NCHC, CTC), jnp.int32),
            pltpu.VMEM((NCHC, CTC), jnp.int32),
            pltpu.VMEM((CTC, 16), jnp.float32),
            pltpu.VMEM((CTC, 16), jnp.float32),
            pltpu.VMEM((CTC, H), jnp.float32),
            pltpu.VMEM((CTC, H), jnp.float32),
            pltpu.VMEM((CTC, H), jnp.float32),
            pltpu.VMEM((CTC, H), jnp.float32),
            pltpu.VMEM((CTC, H), jnp.float32),
            pltpu.SemaphoreType.DMA,
            pltpu.SemaphoreType.DMA,
            pltpu.SemaphoreType.DMA,
            pltpu.SemaphoreType.DMA,
        ],
    )
    return kern(ys,
                d0.reshape(NW, NCHC, CTC), d1.reshape(NW, NCHC, CTC),
                w0.reshape(NW, NCHC, CTC, 16), w1.reshape(NW, NCHC, CTC, 16))


@jax.jit
def kernel(x, gate_w, gate_b, fc1_w, fc1_b, fc2_w, fc2_b):
    x_flat = x.reshape(N, H)
    s0, s1, w0, w1 = _router(x_flat, gate_w, gate_b)
    sel01 = jnp.concatenate(
        [s0.reshape(256, 128), s1.reshape(256, 128)], axis=0)
    dest, be = _sortmeta(sel01)
    d0 = dest[:256].reshape(N)
    d1 = dest[256:].reshape(N)
    block_expert = be.reshape(512)[:GBLK]
    xs = _dispatch(x_flat, d0, d1)
    ys = _ffn(xs, block_expert, fc1_w, fc1_b, fc2_w, fc2_b)
    out = _combine(ys, d0, d1, w0, w1)
    return out.reshape(B, S, H)
